# baseline (speedup 1.0000x reference)
import functools

import jax
import jax.numpy as jnp
from jax import lax
from jax.experimental import pallas as pl
from jax.experimental.pallas import tpu as pltpu
from jax.experimental.pallas import tpu_sc as plsc

N = 50000
E = 800000
G = 400
D_IN = 8
HID = 64
FH = 32
TDIM = 128
NCLS = 2
NCORE = 2
NSUB = 16
NP = 51200
NP2 = 50176
CH = 128
E2 = E + N
SB = 16
NCHUNK = 416
E2P = NSUB * NCHUNK * CH
NSBLK = NCHUNK // SB

_SC_PARAMS = pltpu.CompilerParams(use_tc_tiling_on_sc=False)
_mesh = plsc.VectorSubcoreMesh(core_axis_name="c", subcore_axis_name="s")

_DC = 196
_DW = 28
EP_DEG = NCORE * NSUB * _DC * CH


@functools.partial(
    pl.kernel,
    out_type=jax.ShapeDtypeStruct((NCORE, NP), jnp.float32),
    mesh=_mesh,
    compiler_params=_SC_PARAMS,
    scratch_types=[
        pltpu.VMEM_SHARED((NP,), jnp.float32),
        pltpu.VMEM((_DC, CH), jnp.int32),
        pltpu.VMEM((_DC, CH), jnp.float32),
        pltpu.SemaphoreType.DMA,
    ],
)
def _deg_kernel(dst_hbm, ew_hbm, zeros_hbm, out_hbm, acc_sh, didx_b, w_b, sem):
    c = lax.axis_index("c")
    s = lax.axis_index("s")
    pltpu.sync_copy(zeros_hbm, acc_sh.at[pl.ds(s * (NP // NSUB), NP // NSUB)])
    plsc.subcore_barrier()

    row0 = (c * NSUB + s) * _DC
    pltpu.sync_copy(dst_hbm.at[pl.ds(row0, _DC), :], didx_b)
    pltpu.sync_copy(ew_hbm.at[pl.ds(row0, _DC), :], w_b)

    def _fire(wv):
        for j in range(wv * _DW, (wv + 1) * _DW):
            pltpu.async_copy(w_b.at[j], acc_sh.at[didx_b.at[j]], sem, add=True)

    def _drain(wv):
        for j in range(wv * _DW, (wv + 1) * _DW):
            pltpu.make_async_copy(w_b.at[j], acc_sh.at[didx_b.at[j]], sem).wait()

    _fire(0)
    for wv in range(1, _DC // _DW):
        _fire(wv)
        _drain(wv - 1)
    _drain(_DC // _DW - 1)

    plsc.subcore_barrier()
    sl = pl.ds(s * (NP // NSUB), NP // NSUB)
    pltpu.sync_copy(acc_sh.at[sl], out_hbm.at[c].at[sl])


_NR = NP2 // NSUB

_BCAST_DNUMS = lax.GatherDimensionNumbers(
    offset_dims=(), collapsed_slice_dims=(0,), start_index_map=(0,))


def _lane_bcast(vec, e):
    return lax.gather(vec, jnp.full((16, 1), e, jnp.int32), _BCAST_DNUMS, (1,),
                      mode=lax.GatherScatterMode.PROMISE_IN_BOUNDS)


def _scale_rows(w_ref, rows_ref, nedge):
    def grp(g, _):
        wv = w_ref[pl.ds(g * 16, 16)]
        for e in range(16):
            ei = g * 16 + e
            bc = _lane_bcast(wv, e)
            for f in range(FH // 16):
                r = rows_ref[ei, pl.ds(f * 16, 16)]
                rows_ref[ei, pl.ds(f * 16, 16)] = r * bc
        return 0

    lax.fori_loop(0, nedge // 16, grp, 0)


@functools.partial(
    pl.kernel,
    out_type=jax.ShapeDtypeStruct((NCORE, NP2, FH), jnp.float32),
    mesh=_mesh,
    compiler_params=_SC_PARAMS,
    scratch_types=[
        pltpu.VMEM_SHARED((NP2, FH), jnp.float32),
        pltpu.VMEM((SB, CH), jnp.int32),
        pltpu.VMEM((SB, CH), jnp.int32),
        pltpu.VMEM((SB, CH), jnp.float32),
        pltpu.VMEM((CH, FH), jnp.float32),
        pltpu.VMEM((CH, FH), jnp.float32),
        pltpu.VMEM((CH, FH), jnp.float32),
        pltpu.SemaphoreType.DMA,
        pltpu.SemaphoreType.DMA,
        pltpu.SemaphoreType.DMA,
        pltpu.SemaphoreType.DMA,
        pltpu.SemaphoreType.DMA,
        pltpu.SemaphoreType.DMA,
    ],
)
def _mp_kernel(src_hbm, dst_hbm, ew_hbm, ta_hbm, tb_hbm, zeros2_hbm,
               out_hbm,
               acc_sh, sidx_b, didx_b, w_b, rows0_v, rows1_v, rows2_v,
               gs0, gs1, gs2, ss0, ss1, ss2):
    c = lax.axis_index("c")
    s = lax.axis_index("s")
    rows = (rows0_v, rows1_v, rows2_v)
    gsems = (gs0, gs1, gs2)
    ssems = (ss0, ss1, ss2)

    pltpu.sync_copy(zeros2_hbm, acc_sh.at[pl.ds(s * _NR, _NR), :])
    plsc.subcore_barrier()

    def _gather(j, buf):
        @pl.when(c == 0)
        def _():
            pltpu.async_copy(ta_hbm.at[sidx_b.at[j]], rows[buf], gsems[buf])

        @pl.when(c == 1)
        def _():
            pltpu.async_copy(tb_hbm.at[sidx_b.at[j]], rows[buf], gsems[buf])

    def _gwait(j, buf):
        pltpu.make_async_copy(ta_hbm.at[sidx_b.at[j]], rows[buf], gsems[buf]).wait()

    def _swait(j, buf):
        pltpu.make_async_copy(rows[buf], acc_sh.at[didx_b.at[j]], ssems[buf]).wait()

    def sblock(sb, _):
        row0 = s * NCHUNK + sb * SB
        pltpu.sync_copy(src_hbm.at[pl.ds(row0, SB), :], sidx_b)
        pltpu.sync_copy(dst_hbm.at[pl.ds(row0, SB), :], didx_b)
        pltpu.sync_copy(ew_hbm.at[pl.ds(row0, SB), :], w_b)
        _gather(0, 0)
        _gather(1, 1)
        for j in range(SB):
            buf = j % 3
            if j + 2 < SB:
                nxt = (j + 2) % 3
                if j >= 1:
                    _swait(j - 1, nxt)
                _gather(j + 2, nxt)
            _gwait(j, buf)
            _scale_rows(w_b.at[j], rows[buf], CH)
            pltpu.async_copy(rows[buf], acc_sh.at[didx_b.at[j]], ssems[buf], add=True)
        for j in range(SB - 3, SB):
            _swait(j, j % 3)
        return 0

    lax.fori_loop(0, NSBLK, sblock, 0)
    plsc.subcore_barrier()

    sl = pl.ds(s * _NR, _NR)
    pltpu.sync_copy(acc_sh.at[sl, :], out_hbm.at[c].at[sl, :])


def _prep1_body(x_ref, degp_ref, w1_ref, ta_ref, tb_ref, dv_ref):
    dv = lax.rsqrt(degp_ref[0] + degp_ref[1] + 1.0)
    dv_ref[...] = dv
    xs = x_ref[...] * dv
    h = jnp.dot(xs, w1_ref[...], preferred_element_type=jnp.float32)
    ta_ref[...] = h[:, :FH]
    tb_ref[...] = h[:, FH:]


_RB = 5000


def _combine_body(sp_ref, dv_ref, b_ref, out_ref, stats_ref):
    i = pl.program_id(0)
    dv = dv_ref[...]
    z0 = sp_ref[0] * dv + b_ref[0]
    z1 = sp_ref[1] * dv + b_ref[1]
    out_ref[:, :FH] = z0
    out_ref[:, FH:] = z1

    @pl.when(i == 0)
    def _():
        stats_ref[...] = jnp.zeros((2, HID), jnp.float32)

    stats_ref[0] = stats_ref[0] + jnp.concatenate(
        [jnp.sum(z0, axis=0), jnp.sum(z1, axis=0)])
    stats_ref[1] = stats_ref[1] + jnp.concatenate(
        [jnp.sum(z0 * z0, axis=0), jnp.sum(z1 * z1, axis=0)])


def _combine_call(s, dinv_col, b):
    return pl.pallas_call(
        _combine_body,
        grid=(N // _RB,),
        in_specs=[pl.BlockSpec((NCORE, _RB, FH), lambda i: (0, i, 0)),
                  pl.BlockSpec((_RB, 1), lambda i: (i, 0)),
                  pl.BlockSpec((NCORE, FH), lambda i: (0, 0))],
        out_specs=[pl.BlockSpec((_RB, HID), lambda i: (i, 0)),
                   pl.BlockSpec((2, HID), lambda i: (0, 0))],
        out_shape=(jax.ShapeDtypeStruct((N, HID), jnp.float32),
                   jax.ShapeDtypeStruct((2, HID), jnp.float32)),
    )(s, dinv_col, b)


def _bnmm_body(h_ref, stats_ref, g_ref, be_ref, dv_ref, w2_ref, ta_ref, tb_ref):
    m = stats_ref[0] / float(N)
    v = stats_ref[1] / float(N) - m * m
    scale = lax.rsqrt(v + 1e-5) * g_ref[...]
    z = jnp.maximum(h_ref[...] * scale + (be_ref[...] - m * scale), 0.0)
    z = z * dv_ref[...]
    h2 = jnp.dot(z, w2_ref[...], preferred_element_type=jnp.float32)
    ta_ref[...] = h2[:, :FH]
    tb_ref[...] = h2[:, FH:]


def _bnmm_call(h, stats, g, be, dinv_col, W2):
    return pl.pallas_call(
        _bnmm_body,
        grid=(N // _RB,),
        in_specs=[pl.BlockSpec((_RB, HID), lambda i: (i, 0)),
                  pl.BlockSpec((2, HID), lambda i: (0, 0)),
                  pl.BlockSpec((HID,), lambda i: (0,)),
                  pl.BlockSpec((HID,), lambda i: (0,)),
                  pl.BlockSpec((_RB, 1), lambda i: (i, 0)),
                  pl.BlockSpec((HID, HID), lambda i: (0, 0))],
        out_specs=[pl.BlockSpec((_RB, FH), lambda i: (i, 0)),
                   pl.BlockSpec((_RB, FH), lambda i: (i, 0))],
        out_shape=(jax.ShapeDtypeStruct((NP2, FH), jnp.float32),
                   jax.ShapeDtypeStruct((NP2, FH), jnp.float32)),
    )(h, stats, g, be, dinv_col, W2)


_POOL_CHUNK = 2000


def _poolgru_body(h_ref, stats_ref, g_ref, be_ref, bids_ref,
                  wih_ref, whh_ref, bih_ref, bhh_ref,
                  wc1_ref, bc1_ref, wc2_ref, bc2_ref, out_ref, gi_ref):
    m = stats_ref[0] / float(N)
    v = stats_ref[1] / float(N) - m * m
    scale = lax.rsqrt(v + 1e-5) * g_ref[...]
    shift = be_ref[...] - m * scale

    def pstep(k, carry):
        acc, cnt = carry
        off = pl.multiple_of(k * _POOL_CHUNK, 8)
        zc = jnp.maximum(h_ref[pl.ds(off, _POOL_CHUNK), :] * scale + shift, 0.0)
        bi = bids_ref[k]
        onehot = (bi[:, None] == lax.broadcasted_iota(jnp.int32, (1, G), 1)).astype(jnp.float32)
        acc = acc + lax.dot_general(onehot, zc, (((0,), (0,)), ((), ())),
                                    preferred_element_type=jnp.float32)
        cnt = cnt + jnp.sum(onehot, axis=0)
        return acc, cnt

    sums, cnt = lax.fori_loop(0, N // _POOL_CHUNK, pstep,
                              (jnp.zeros((G, HID), jnp.float32), jnp.zeros((G,), jnp.float32)))
    pooled = sums / jnp.maximum(cnt.reshape(G, 1), 1.0)
    gi_ref[...] = lax.dot_general(pooled, wih_ref[...], (((1,), (1,)), ((), ())),
                                  preferred_element_type=jnp.float32) + bih_ref[...]

    def step(t, h):
        gi = gi_ref[pl.ds(t, 1), :]
        gh = lax.dot_general(h, whh_ref[...], (((1,), (1,)), ((), ())),
                             preferred_element_type=jnp.float32) + bhh_ref[...]
        ir = gi[:, :TDIM]
        iz = gi[:, TDIM:2 * TDIM]
        inn = gi[:, 2 * TDIM:]
        hr = gh[:, :TDIM]
        hz = gh[:, TDIM:2 * TDIM]
        hn = gh[:, 2 * TDIM:]
        r = jax.nn.sigmoid(ir + hr)
        z = jax.nn.sigmoid(iz + hz)
        ng = jnp.tanh(inn + r * hn)
        return (1.0 - z) * ng + z * h

    h = lax.fori_loop(0, G, step, jnp.zeros((1, TDIM), jnp.float32))
    z1 = jnp.maximum(jnp.dot(h, wc1_ref[...], preferred_element_type=jnp.float32)
                     + bc1_ref[...], 0.0)
    out_ref[...] = jnp.dot(z1, wc2_ref[...], preferred_element_type=jnp.float32) + bc2_ref[...]


def _tc_call(body, out_shapes, *args, scratch_shapes=()):
    return pl.pallas_call(body, out_shape=out_shapes,
                          scratch_shapes=list(scratch_shapes))(*args)


@jax.jit
def kernel(x, edge_index, edge_weight, batch_ids, t, W1, b1, W2, b2, g1, be1,
           g2, be2, Wih, Whh, bih, bhh, Wc1, bc1, Wc2, bc2):
    src = edge_index[0]
    dst = edge_index[1]
    iota_n = jnp.arange(N, dtype=jnp.int32)
    padi = jnp.zeros((E2P - E2,), jnp.int32)
    src2 = jnp.concatenate([src, iota_n, padi]).reshape(E2P // CH, CH)
    dst2 = jnp.concatenate([dst, iota_n, padi]).reshape(E2P // CH, CH)
    ew2 = jnp.concatenate([edge_weight, jnp.ones((N,), jnp.float32),
                           jnp.zeros((E2P - E2,), jnp.float32)]).reshape(E2P // CH, CH)
    x_pad = jnp.pad(x, ((0, NP2 - N), (0, 0)))
    dstd = jnp.pad(dst, (0, EP_DEG - E)).reshape(EP_DEG // CH, CH)
    ewd = jnp.pad(edge_weight, (0, EP_DEG - E)).reshape(EP_DEG // CH, CH)

    zeros1 = jnp.zeros((NP // NSUB,), jnp.float32)
    zeros2 = jnp.zeros((_NR, FH), jnp.float32)

    degp = _deg_kernel(dstd, ewd, zeros1)
    degp_c = degp[:, :NP2].reshape(NCORE, NP2, 1)

    t1a, t1b, dinv_col = pl.pallas_call(
        _prep1_body,
        grid=(8,),
        in_specs=[pl.BlockSpec((NP2 // 8, D_IN), lambda i: (i, 0)),
                  pl.BlockSpec((NCORE, NP2 // 8, 1), lambda i: (0, i, 0)),
                  pl.BlockSpec((D_IN, HID), lambda i: (0, 0))],
        out_specs=[pl.BlockSpec((NP2 // 8, FH), lambda i: (i, 0)),
                   pl.BlockSpec((NP2 // 8, FH), lambda i: (i, 0)),
                   pl.BlockSpec((NP2 // 8, 1), lambda i: (i, 0))],
        out_shape=(jax.ShapeDtypeStruct((NP2, FH), jnp.float32),
                   jax.ShapeDtypeStruct((NP2, FH), jnp.float32),
                   jax.ShapeDtypeStruct((NP2, 1), jnp.float32)),
    )(x_pad, degp_c, W1)

    s1 = _mp_kernel(src2, dst2, ew2, t1a, t1b, zeros2)

    h1, stats1 = _combine_call(s1, dinv_col, b1.reshape(2, FH))

    t2a, t2b = _bnmm_call(h1, stats1, g1, be1, dinv_col, W2)

    s2 = _mp_kernel(src2, dst2, ew2, t2a, t2b, zeros2)

    h2, stats2 = _combine_call(s2, dinv_col, b2.reshape(2, FH))

    logits = _tc_call(
        _poolgru_body,
        jax.ShapeDtypeStruct((1, NCLS), jnp.float32),
        h2, stats2, g2, be2, batch_ids.reshape(N // _POOL_CHUNK, _POOL_CHUNK),
        Wih, Whh, bih, bhh, Wc1, bc1, Wc2, bc2,
        scratch_shapes=[pltpu.VMEM((G, 3 * TDIM), jnp.float32)])
    return logits


# speedup vs baseline: 1.0515x; 1.0515x over previous
import functools

import jax
import jax.numpy as jnp
from jax import lax
from jax.experimental import pallas as pl
from jax.experimental.pallas import tpu as pltpu
from jax.experimental.pallas import tpu_sc as plsc

N = 50000
E = 800000
G = 400
D_IN = 8
HID = 64
FH = 32
TDIM = 128
NCLS = 2
NCORE = 2
NSUB = 16
NP = 51200
NP2 = 50176
CH = 128
E2 = E + N
SB = 32
NCHUNK = 416
E2P = NSUB * NCHUNK * CH
NSBLK = NCHUNK // SB

_SC_PARAMS = pltpu.CompilerParams(use_tc_tiling_on_sc=False)
_mesh = plsc.VectorSubcoreMesh(core_axis_name="c", subcore_axis_name="s")

_DC = 196
_DW = 28
EP_DEG = NCORE * NSUB * _DC * CH


@functools.partial(
    pl.kernel,
    out_type=jax.ShapeDtypeStruct((NCORE, NP), jnp.float32),
    mesh=_mesh,
    compiler_params=_SC_PARAMS,
    scratch_types=[
        pltpu.VMEM_SHARED((NP,), jnp.float32),
        pltpu.VMEM((_DC, CH), jnp.int32),
        pltpu.VMEM((_DC, CH), jnp.float32),
        pltpu.SemaphoreType.DMA,
    ],
)
def _deg_kernel(dst_hbm, ew_hbm, zeros_hbm, out_hbm, acc_sh, didx_b, w_b, sem):
    c = lax.axis_index("c")
    s = lax.axis_index("s")
    pltpu.sync_copy(zeros_hbm, acc_sh.at[pl.ds(s * (NP // NSUB), NP // NSUB)])
    plsc.subcore_barrier()

    row0 = (c * NSUB + s) * _DC
    pltpu.sync_copy(dst_hbm.at[pl.ds(row0, _DC), :], didx_b)
    pltpu.sync_copy(ew_hbm.at[pl.ds(row0, _DC), :], w_b)

    def _fire(wv):
        for j in range(wv * _DW, (wv + 1) * _DW):
            pltpu.async_copy(w_b.at[j], acc_sh.at[didx_b.at[j]], sem, add=True)

    def _drain(wv):
        for j in range(wv * _DW, (wv + 1) * _DW):
            pltpu.make_async_copy(w_b.at[j], acc_sh.at[didx_b.at[j]], sem).wait()

    _fire(0)
    for wv in range(1, _DC // _DW):
        _fire(wv)
        _drain(wv - 1)
    _drain(_DC // _DW - 1)

    plsc.subcore_barrier()
    sl = pl.ds(s * (NP // NSUB), NP // NSUB)
    pltpu.sync_copy(acc_sh.at[sl], out_hbm.at[c].at[sl])


_NR = NP2 // NSUB

_BCAST_DNUMS = lax.GatherDimensionNumbers(
    offset_dims=(), collapsed_slice_dims=(0,), start_index_map=(0,))


def _lane_bcast(vec, e):
    return lax.gather(vec, jnp.full((16, 1), e, jnp.int32), _BCAST_DNUMS, (1,),
                      mode=lax.GatherScatterMode.PROMISE_IN_BOUNDS)


def _scale_rows(w_ref, rows_ref, nedge):
    def grp(g, _):
        wv = w_ref[pl.ds(g * 16, 16)]
        for e in range(16):
            ei = g * 16 + e
            bc = _lane_bcast(wv, e)
            for f in range(FH // 16):
                r = rows_ref[ei, pl.ds(f * 16, 16)]
                rows_ref[ei, pl.ds(f * 16, 16)] = r * bc
        return 0

    lax.fori_loop(0, nedge // 16, grp, 0)


@functools.partial(
    pl.kernel,
    out_type=jax.ShapeDtypeStruct((NCORE, NP2, FH), jnp.float32),
    mesh=_mesh,
    compiler_params=_SC_PARAMS,
    scratch_types=[
        pltpu.VMEM_SHARED((NP2, FH), jnp.float32),
        pltpu.VMEM((SB, CH), jnp.int32),
        pltpu.VMEM((SB, CH), jnp.int32),
        pltpu.VMEM((SB, CH), jnp.float32),
        pltpu.VMEM((CH, FH), jnp.float32),
        pltpu.VMEM((CH, FH), jnp.float32),
        pltpu.VMEM((CH, FH), jnp.float32),
        pltpu.SemaphoreType.DMA,
        pltpu.SemaphoreType.DMA,
        pltpu.SemaphoreType.DMA,
        pltpu.SemaphoreType.DMA,
        pltpu.SemaphoreType.DMA,
        pltpu.SemaphoreType.DMA,
    ],
)
def _mp_kernel(src_hbm, dst_hbm, ew_hbm, ta_hbm, tb_hbm, zeros2_hbm,
               out_hbm,
               acc_sh, sidx_b, didx_b, w_b, rows0_v, rows1_v, rows2_v,
               gs0, gs1, gs2, ss0, ss1, ss2):
    c = lax.axis_index("c")
    s = lax.axis_index("s")
    rows = (rows0_v, rows1_v, rows2_v)
    gsems = (gs0, gs1, gs2)
    ssems = (ss0, ss1, ss2)

    pltpu.sync_copy(zeros2_hbm, acc_sh.at[pl.ds(s * _NR, _NR), :])
    plsc.subcore_barrier()

    def _gather(j, buf):
        @pl.when(c == 0)
        def _():
            pltpu.async_copy(ta_hbm.at[sidx_b.at[j]], rows[buf], gsems[buf])

        @pl.when(c == 1)
        def _():
            pltpu.async_copy(tb_hbm.at[sidx_b.at[j]], rows[buf], gsems[buf])

    def _gwait(j, buf):
        pltpu.make_async_copy(ta_hbm.at[sidx_b.at[j]], rows[buf], gsems[buf]).wait()

    def _swait(j, buf):
        pltpu.make_async_copy(rows[buf], acc_sh.at[didx_b.at[j]], ssems[buf]).wait()

    def sblock(sb, _):
        row0 = s * NCHUNK + sb * SB
        pltpu.sync_copy(src_hbm.at[pl.ds(row0, SB), :], sidx_b)
        pltpu.sync_copy(dst_hbm.at[pl.ds(row0, SB), :], didx_b)
        pltpu.sync_copy(ew_hbm.at[pl.ds(row0, SB), :], w_b)
        _gather(0, 0)
        _gather(1, 1)
        for j in range(SB):
            buf = j % 3
            if j + 2 < SB:
                nxt = (j + 2) % 3
                if j >= 1:
                    _swait(j - 1, nxt)
                _gather(j + 2, nxt)
            _gwait(j, buf)
            _scale_rows(w_b.at[j], rows[buf], CH)
            pltpu.async_copy(rows[buf], acc_sh.at[didx_b.at[j]], ssems[buf], add=True)
        for j in range(SB - 3, SB):
            _swait(j, j % 3)
        return 0

    lax.fori_loop(0, NSBLK, sblock, 0)
    plsc.subcore_barrier()

    sl = pl.ds(s * _NR, _NR)
    pltpu.sync_copy(acc_sh.at[sl, :], out_hbm.at[c].at[sl, :])


def _prep1_body(x_ref, degp_ref, w1_ref, ta_ref, tb_ref, dv_ref):
    dv = lax.rsqrt(degp_ref[0] + degp_ref[1] + 1.0)
    dv_ref[...] = dv
    xs = x_ref[...] * dv
    h = jnp.dot(xs, w1_ref[...], preferred_element_type=jnp.float32)
    ta_ref[...] = h[:, :FH]
    tb_ref[...] = h[:, FH:]


_RB = 5000


def _combine_body(sp_ref, dv_ref, b_ref, out_ref, stats_ref):
    i = pl.program_id(0)
    dv = dv_ref[...]
    z0 = sp_ref[0] * dv + b_ref[0]
    z1 = sp_ref[1] * dv + b_ref[1]
    out_ref[:, :FH] = z0
    out_ref[:, FH:] = z1

    @pl.when(i == 0)
    def _():
        stats_ref[...] = jnp.zeros((2, HID), jnp.float32)

    stats_ref[0] = stats_ref[0] + jnp.concatenate(
        [jnp.sum(z0, axis=0), jnp.sum(z1, axis=0)])
    stats_ref[1] = stats_ref[1] + jnp.concatenate(
        [jnp.sum(z0 * z0, axis=0), jnp.sum(z1 * z1, axis=0)])


def _combine_call(s, dinv_col, b):
    return pl.pallas_call(
        _combine_body,
        grid=(N // _RB,),
        in_specs=[pl.BlockSpec((NCORE, _RB, FH), lambda i: (0, i, 0)),
                  pl.BlockSpec((_RB, 1), lambda i: (i, 0)),
                  pl.BlockSpec((NCORE, FH), lambda i: (0, 0))],
        out_specs=[pl.BlockSpec((_RB, HID), lambda i: (i, 0)),
                   pl.BlockSpec((2, HID), lambda i: (0, 0))],
        out_shape=(jax.ShapeDtypeStruct((N, HID), jnp.float32),
                   jax.ShapeDtypeStruct((2, HID), jnp.float32)),
    )(s, dinv_col, b)


def _bnmm_body(h_ref, stats_ref, g_ref, be_ref, dv_ref, w2_ref, ta_ref, tb_ref):
    m = stats_ref[0] / float(N)
    v = stats_ref[1] / float(N) - m * m
    scale = lax.rsqrt(v + 1e-5) * g_ref[...]
    z = jnp.maximum(h_ref[...] * scale + (be_ref[...] - m * scale), 0.0)
    z = z * dv_ref[...]
    h2 = jnp.dot(z, w2_ref[...], preferred_element_type=jnp.float32)
    ta_ref[...] = h2[:, :FH]
    tb_ref[...] = h2[:, FH:]


def _bnmm_call(h, stats, g, be, dinv_col, W2):
    return pl.pallas_call(
        _bnmm_body,
        grid=(N // _RB,),
        in_specs=[pl.BlockSpec((_RB, HID), lambda i: (i, 0)),
                  pl.BlockSpec((2, HID), lambda i: (0, 0)),
                  pl.BlockSpec((HID,), lambda i: (0,)),
                  pl.BlockSpec((HID,), lambda i: (0,)),
                  pl.BlockSpec((_RB, 1), lambda i: (i, 0)),
                  pl.BlockSpec((HID, HID), lambda i: (0, 0))],
        out_specs=[pl.BlockSpec((_RB, FH), lambda i: (i, 0)),
                   pl.BlockSpec((_RB, FH), lambda i: (i, 0))],
        out_shape=(jax.ShapeDtypeStruct((NP2, FH), jnp.float32),
                   jax.ShapeDtypeStruct((NP2, FH), jnp.float32)),
    )(h, stats, g, be, dinv_col, W2)


_POOL_CHUNK = 2000


def _poolgru_body(h_ref, stats_ref, g_ref, be_ref, bids_ref,
                  wih_ref, whh_ref, bih_ref, bhh_ref,
                  wc1_ref, bc1_ref, wc2_ref, bc2_ref, out_ref, gi_ref):
    m = stats_ref[0] / float(N)
    v = stats_ref[1] / float(N) - m * m
    scale = lax.rsqrt(v + 1e-5) * g_ref[...]
    shift = be_ref[...] - m * scale

    def pstep(k, carry):
        acc, cnt = carry
        off = pl.multiple_of(k * _POOL_CHUNK, 8)
        zc = jnp.maximum(h_ref[pl.ds(off, _POOL_CHUNK), :] * scale + shift, 0.0)
        bi = bids_ref[k]
        onehot = (bi[:, None] == lax.broadcasted_iota(jnp.int32, (1, G), 1)).astype(jnp.float32)
        acc = acc + lax.dot_general(onehot, zc, (((0,), (0,)), ((), ())),
                                    preferred_element_type=jnp.float32)
        cnt = cnt + jnp.sum(onehot, axis=0)
        return acc, cnt

    sums, cnt = lax.fori_loop(0, N // _POOL_CHUNK, pstep,
                              (jnp.zeros((G, HID), jnp.float32), jnp.zeros((G,), jnp.float32)))
    pooled = sums / jnp.maximum(cnt.reshape(G, 1), 1.0)
    gi_ref[...] = lax.dot_general(pooled, wih_ref[...], (((1,), (1,)), ((), ())),
                                  preferred_element_type=jnp.float32) + bih_ref[...]

    def step(t, h):
        gi = gi_ref[pl.ds(t, 1), :]
        gh = lax.dot_general(h, whh_ref[...], (((1,), (1,)), ((), ())),
                             preferred_element_type=jnp.float32) + bhh_ref[...]
        ir = gi[:, :TDIM]
        iz = gi[:, TDIM:2 * TDIM]
        inn = gi[:, 2 * TDIM:]
        hr = gh[:, :TDIM]
        hz = gh[:, TDIM:2 * TDIM]
        hn = gh[:, 2 * TDIM:]
        r = jax.nn.sigmoid(ir + hr)
        z = jax.nn.sigmoid(iz + hz)
        ng = jnp.tanh(inn + r * hn)
        return (1.0 - z) * ng + z * h

    h = lax.fori_loop(0, G, step, jnp.zeros((1, TDIM), jnp.float32))
    z1 = jnp.maximum(jnp.dot(h, wc1_ref[...], preferred_element_type=jnp.float32)
                     + bc1_ref[...], 0.0)
    out_ref[...] = jnp.dot(z1, wc2_ref[...], preferred_element_type=jnp.float32) + bc2_ref[...]


def _tc_call(body, out_shapes, *args, scratch_shapes=()):
    return pl.pallas_call(body, out_shape=out_shapes,
                          scratch_shapes=list(scratch_shapes))(*args)


@jax.jit
def kernel(x, edge_index, edge_weight, batch_ids, t, W1, b1, W2, b2, g1, be1,
           g2, be2, Wih, Whh, bih, bhh, Wc1, bc1, Wc2, bc2):
    src = edge_index[0]
    dst = edge_index[1]
    iota_n = jnp.arange(N, dtype=jnp.int32)
    padi = jnp.zeros((E2P - E2,), jnp.int32)
    src2 = jnp.concatenate([src, iota_n, padi]).reshape(E2P // CH, CH)
    dst2 = jnp.concatenate([dst, iota_n, padi]).reshape(E2P // CH, CH)
    ew2 = jnp.concatenate([edge_weight, jnp.ones((N,), jnp.float32),
                           jnp.zeros((E2P - E2,), jnp.float32)]).reshape(E2P // CH, CH)
    x_pad = jnp.pad(x, ((0, NP2 - N), (0, 0)))
    dstd = jnp.pad(dst, (0, EP_DEG - E)).reshape(EP_DEG // CH, CH)
    ewd = jnp.pad(edge_weight, (0, EP_DEG - E)).reshape(EP_DEG // CH, CH)

    zeros1 = jnp.zeros((NP // NSUB,), jnp.float32)
    zeros2 = jnp.zeros((_NR, FH), jnp.float32)

    degp = _deg_kernel(dstd, ewd, zeros1)
    degp_c = degp[:, :NP2].reshape(NCORE, NP2, 1)

    t1a, t1b, dinv_col = pl.pallas_call(
        _prep1_body,
        grid=(8,),
        in_specs=[pl.BlockSpec((NP2 // 8, D_IN), lambda i: (i, 0)),
                  pl.BlockSpec((NCORE, NP2 // 8, 1), lambda i: (0, i, 0)),
                  pl.BlockSpec((D_IN, HID), lambda i: (0, 0))],
        out_specs=[pl.BlockSpec((NP2 // 8, FH), lambda i: (i, 0)),
                   pl.BlockSpec((NP2 // 8, FH), lambda i: (i, 0)),
                   pl.BlockSpec((NP2 // 8, 1), lambda i: (i, 0))],
        out_shape=(jax.ShapeDtypeStruct((NP2, FH), jnp.float32),
                   jax.ShapeDtypeStruct((NP2, FH), jnp.float32),
                   jax.ShapeDtypeStruct((NP2, 1), jnp.float32)),
    )(x_pad, degp_c, W1)

    s1 = _mp_kernel(src2, dst2, ew2, t1a, t1b, zeros2)

    h1, stats1 = _combine_call(s1, dinv_col, b1.reshape(2, FH))

    t2a, t2b = _bnmm_call(h1, stats1, g1, be1, dinv_col, W2)

    s2 = _mp_kernel(src2, dst2, ew2, t2a, t2b, zeros2)

    h2, stats2 = _combine_call(s2, dinv_col, b2.reshape(2, FH))

    logits = _tc_call(
        _poolgru_body,
        jax.ShapeDtypeStruct((1, NCLS), jnp.float32),
        h2, stats2, g2, be2, batch_ids.reshape(N // _POOL_CHUNK, _POOL_CHUNK),
        Wih, Whh, bih, bhh, Wc1, bc1, Wc2, bc2,
        scratch_shapes=[pltpu.VMEM((G, 3 * TDIM), jnp.float32)])
    return logits


# speedup vs baseline: 1.0566x; 1.0049x over previous
import functools

import jax
import jax.numpy as jnp
from jax import lax
from jax.experimental import pallas as pl
from jax.experimental.pallas import tpu as pltpu
from jax.experimental.pallas import tpu_sc as plsc

N = 50000
E = 800000
G = 400
D_IN = 8
HID = 64
FH = 32
TDIM = 128
NCLS = 2
NCORE = 2
NSUB = 16
NP = 51200
NP2 = 50176
CH = 128
E2 = E + N
SB = 32
NCHUNK = 416
E2P = NSUB * NCHUNK * CH
NSBLK = NCHUNK // SB

_SC_PARAMS = pltpu.CompilerParams(use_tc_tiling_on_sc=False)
_mesh = plsc.VectorSubcoreMesh(core_axis_name="c", subcore_axis_name="s")

_DC = 196
_DW = 28
EP_DEG = NCORE * NSUB * _DC * CH


@functools.partial(
    pl.kernel,
    out_type=jax.ShapeDtypeStruct((NCORE, NP), jnp.float32),
    mesh=_mesh,
    compiler_params=_SC_PARAMS,
    scratch_types=[
        pltpu.VMEM_SHARED((NP,), jnp.float32),
        pltpu.VMEM((_DC, CH), jnp.int32),
        pltpu.VMEM((_DC, CH), jnp.float32),
        pltpu.SemaphoreType.DMA,
    ],
)
def _deg_kernel(dst_hbm, ew_hbm, zeros_hbm, out_hbm, acc_sh, didx_b, w_b, sem):
    c = lax.axis_index("c")
    s = lax.axis_index("s")
    pltpu.sync_copy(zeros_hbm, acc_sh.at[pl.ds(s * (NP // NSUB), NP // NSUB)])
    plsc.subcore_barrier()

    row0 = (c * NSUB + s) * _DC
    pltpu.sync_copy(dst_hbm.at[pl.ds(row0, _DC), :], didx_b)
    pltpu.sync_copy(ew_hbm.at[pl.ds(row0, _DC), :], w_b)

    def _fire(wv):
        for j in range(wv * _DW, (wv + 1) * _DW):
            pltpu.async_copy(w_b.at[j], acc_sh.at[didx_b.at[j]], sem, add=True)

    def _drain(wv):
        for j in range(wv * _DW, (wv + 1) * _DW):
            pltpu.make_async_copy(w_b.at[j], acc_sh.at[didx_b.at[j]], sem).wait()

    _fire(0)
    for wv in range(1, _DC // _DW):
        _fire(wv)
        _drain(wv - 1)
    _drain(_DC // _DW - 1)

    plsc.subcore_barrier()
    sl = pl.ds(s * (NP // NSUB), NP // NSUB)
    pltpu.sync_copy(acc_sh.at[sl], out_hbm.at[c].at[sl])


_NR = NP2 // NSUB

_BCAST_DNUMS = lax.GatherDimensionNumbers(
    offset_dims=(), collapsed_slice_dims=(0,), start_index_map=(0,))


def _lane_bcast(vec, e):
    return lax.gather(vec, jnp.full((16, 1), e, jnp.int32), _BCAST_DNUMS, (1,),
                      mode=lax.GatherScatterMode.PROMISE_IN_BOUNDS)


def _scale_rows(w_ref, rows_ref, nedge):
    def grp(g, _):
        wv = w_ref[pl.ds(g * 16, 16)]
        for e in range(16):
            ei = g * 16 + e
            bc = _lane_bcast(wv, e)
            for f in range(FH // 16):
                r = rows_ref[ei, pl.ds(f * 16, 16)]
                rows_ref[ei, pl.ds(f * 16, 16)] = r * bc
        return 0

    lax.fori_loop(0, nedge // 16, grp, 0)


@functools.partial(
    pl.kernel,
    out_type=jax.ShapeDtypeStruct((NCORE, NP2, FH), jnp.float32),
    mesh=_mesh,
    compiler_params=_SC_PARAMS,
    scratch_types=[
        pltpu.VMEM_SHARED((NP2, FH), jnp.float32),
        pltpu.VMEM((SB, CH), jnp.int32),
        pltpu.VMEM((SB, CH), jnp.int32),
        pltpu.VMEM((SB, CH), jnp.float32),
        pltpu.VMEM((CH, FH), jnp.float32),
        pltpu.VMEM((CH, FH), jnp.float32),
        pltpu.VMEM((CH, FH), jnp.float32),
        pltpu.VMEM((CH, FH), jnp.float32),
        pltpu.SemaphoreType.DMA,
        pltpu.SemaphoreType.DMA,
        pltpu.SemaphoreType.DMA,
        pltpu.SemaphoreType.DMA,
        pltpu.SemaphoreType.DMA,
        pltpu.SemaphoreType.DMA,
        pltpu.SemaphoreType.DMA,
        pltpu.SemaphoreType.DMA,
    ],
)
def _mp_kernel(src_hbm, dst_hbm, ew_hbm, ta_hbm, tb_hbm, zeros2_hbm,
               out_hbm,
               acc_sh, sidx_b, didx_b, w_b, rows0_v, rows1_v, rows2_v, rows3_v,
               gs0, gs1, gs2, gs3, ss0, ss1, ss2, ss3):
    c = lax.axis_index("c")
    s = lax.axis_index("s")
    rows = (rows0_v, rows1_v, rows2_v, rows3_v)
    gsems = (gs0, gs1, gs2, gs3)
    ssems = (ss0, ss1, ss2, ss3)

    pltpu.sync_copy(zeros2_hbm, acc_sh.at[pl.ds(s * _NR, _NR), :])
    plsc.subcore_barrier()

    def _gather(j, buf):
        @pl.when(c == 0)
        def _():
            pltpu.async_copy(ta_hbm.at[sidx_b.at[j]], rows[buf], gsems[buf])

        @pl.when(c == 1)
        def _():
            pltpu.async_copy(tb_hbm.at[sidx_b.at[j]], rows[buf], gsems[buf])

    def _gwait(j, buf):
        pltpu.make_async_copy(ta_hbm.at[sidx_b.at[j]], rows[buf], gsems[buf]).wait()

    def _swait(j, buf):
        pltpu.make_async_copy(rows[buf], acc_sh.at[didx_b.at[j]], ssems[buf]).wait()

    def sblock(sb, _):
        row0 = s * NCHUNK + sb * SB
        pltpu.sync_copy(src_hbm.at[pl.ds(row0, SB), :], sidx_b)
        pltpu.sync_copy(dst_hbm.at[pl.ds(row0, SB), :], didx_b)
        pltpu.sync_copy(ew_hbm.at[pl.ds(row0, SB), :], w_b)
        _gather(0, 0)
        _gather(1, 1)
        _gather(2, 2)
        for j in range(SB):
            buf = j % 4
            if j + 3 < SB:
                nxt = (j + 3) % 4
                if j >= 1:
                    _swait(j - 1, nxt)
                _gather(j + 3, nxt)
            _gwait(j, buf)
            _scale_rows(w_b.at[j], rows[buf], CH)
            pltpu.async_copy(rows[buf], acc_sh.at[didx_b.at[j]], ssems[buf], add=True)
        for j in range(SB - 4, SB):
            _swait(j, j % 4)
        return 0

    lax.fori_loop(0, NSBLK, sblock, 0)
    plsc.subcore_barrier()

    sl = pl.ds(s * _NR, _NR)
    pltpu.sync_copy(acc_sh.at[sl, :], out_hbm.at[c].at[sl, :])


def _prep1_body(x_ref, degp_ref, w1_ref, ta_ref, tb_ref, dv_ref):
    dv = lax.rsqrt(degp_ref[0] + degp_ref[1] + 1.0)
    dv_ref[...] = dv
    xs = x_ref[...] * dv
    h = jnp.dot(xs, w1_ref[...], preferred_element_type=jnp.float32)
    ta_ref[...] = h[:, :FH]
    tb_ref[...] = h[:, FH:]


_RB = 5000


def _combine_body(sp_ref, dv_ref, b_ref, out_ref, stats_ref):
    i = pl.program_id(0)
    dv = dv_ref[...]
    z0 = sp_ref[0] * dv + b_ref[0]
    z1 = sp_ref[1] * dv + b_ref[1]
    out_ref[:, :FH] = z0
    out_ref[:, FH:] = z1

    @pl.when(i == 0)
    def _():
        stats_ref[...] = jnp.zeros((2, HID), jnp.float32)

    stats_ref[0] = stats_ref[0] + jnp.concatenate(
        [jnp.sum(z0, axis=0), jnp.sum(z1, axis=0)])
    stats_ref[1] = stats_ref[1] + jnp.concatenate(
        [jnp.sum(z0 * z0, axis=0), jnp.sum(z1 * z1, axis=0)])


def _combine_call(s, dinv_col, b):
    return pl.pallas_call(
        _combine_body,
        grid=(N // _RB,),
        in_specs=[pl.BlockSpec((NCORE, _RB, FH), lambda i: (0, i, 0)),
                  pl.BlockSpec((_RB, 1), lambda i: (i, 0)),
                  pl.BlockSpec((NCORE, FH), lambda i: (0, 0))],
        out_specs=[pl.BlockSpec((_RB, HID), lambda i: (i, 0)),
                   pl.BlockSpec((2, HID), lambda i: (0, 0))],
        out_shape=(jax.ShapeDtypeStruct((N, HID), jnp.float32),
                   jax.ShapeDtypeStruct((2, HID), jnp.float32)),
    )(s, dinv_col, b)


def _bnmm_body(h_ref, stats_ref, g_ref, be_ref, dv_ref, w2_ref, ta_ref, tb_ref):
    m = stats_ref[0] / float(N)
    v = stats_ref[1] / float(N) - m * m
    scale = lax.rsqrt(v + 1e-5) * g_ref[...]
    z = jnp.maximum(h_ref[...] * scale + (be_ref[...] - m * scale), 0.0)
    z = z * dv_ref[...]
    h2 = jnp.dot(z, w2_ref[...], preferred_element_type=jnp.float32)
    ta_ref[...] = h2[:, :FH]
    tb_ref[...] = h2[:, FH:]


def _bnmm_call(h, stats, g, be, dinv_col, W2):
    return pl.pallas_call(
        _bnmm_body,
        grid=(N // _RB,),
        in_specs=[pl.BlockSpec((_RB, HID), lambda i: (i, 0)),
                  pl.BlockSpec((2, HID), lambda i: (0, 0)),
                  pl.BlockSpec((HID,), lambda i: (0,)),
                  pl.BlockSpec((HID,), lambda i: (0,)),
                  pl.BlockSpec((_RB, 1), lambda i: (i, 0)),
                  pl.BlockSpec((HID, HID), lambda i: (0, 0))],
        out_specs=[pl.BlockSpec((_RB, FH), lambda i: (i, 0)),
                   pl.BlockSpec((_RB, FH), lambda i: (i, 0))],
        out_shape=(jax.ShapeDtypeStruct((NP2, FH), jnp.float32),
                   jax.ShapeDtypeStruct((NP2, FH), jnp.float32)),
    )(h, stats, g, be, dinv_col, W2)


_POOL_CHUNK = 2000


def _poolgru_body(h_ref, stats_ref, g_ref, be_ref, bids_ref,
                  wih_ref, whh_ref, bih_ref, bhh_ref,
                  wc1_ref, bc1_ref, wc2_ref, bc2_ref, out_ref, gi_ref):
    m = stats_ref[0] / float(N)
    v = stats_ref[1] / float(N) - m * m
    scale = lax.rsqrt(v + 1e-5) * g_ref[...]
    shift = be_ref[...] - m * scale

    def pstep(k, carry):
        acc, cnt = carry
        off = pl.multiple_of(k * _POOL_CHUNK, 8)
        zc = jnp.maximum(h_ref[pl.ds(off, _POOL_CHUNK), :] * scale + shift, 0.0)
        bi = bids_ref[k]
        onehot = (bi[:, None] == lax.broadcasted_iota(jnp.int32, (1, G), 1)).astype(jnp.float32)
        acc = acc + lax.dot_general(onehot, zc, (((0,), (0,)), ((), ())),
                                    preferred_element_type=jnp.float32)
        cnt = cnt + jnp.sum(onehot, axis=0)
        return acc, cnt

    sums, cnt = lax.fori_loop(0, N // _POOL_CHUNK, pstep,
                              (jnp.zeros((G, HID), jnp.float32), jnp.zeros((G,), jnp.float32)))
    pooled = sums / jnp.maximum(cnt.reshape(G, 1), 1.0)
    gi_ref[...] = lax.dot_general(pooled, wih_ref[...], (((1,), (1,)), ((), ())),
                                  preferred_element_type=jnp.float32) + bih_ref[...]

    def step(t, h):
        gi = gi_ref[pl.ds(t, 1), :]
        gh = lax.dot_general(h, whh_ref[...], (((1,), (1,)), ((), ())),
                             preferred_element_type=jnp.float32) + bhh_ref[...]
        ir = gi[:, :TDIM]
        iz = gi[:, TDIM:2 * TDIM]
        inn = gi[:, 2 * TDIM:]
        hr = gh[:, :TDIM]
        hz = gh[:, TDIM:2 * TDIM]
        hn = gh[:, 2 * TDIM:]
        r = jax.nn.sigmoid(ir + hr)
        z = jax.nn.sigmoid(iz + hz)
        ng = jnp.tanh(inn + r * hn)
        return (1.0 - z) * ng + z * h

    h = lax.fori_loop(0, G, step, jnp.zeros((1, TDIM), jnp.float32))
    z1 = jnp.maximum(jnp.dot(h, wc1_ref[...], preferred_element_type=jnp.float32)
                     + bc1_ref[...], 0.0)
    out_ref[...] = jnp.dot(z1, wc2_ref[...], preferred_element_type=jnp.float32) + bc2_ref[...]


def _tc_call(body, out_shapes, *args, scratch_shapes=()):
    return pl.pallas_call(body, out_shape=out_shapes,
                          scratch_shapes=list(scratch_shapes))(*args)


@jax.jit
def kernel(x, edge_index, edge_weight, batch_ids, t, W1, b1, W2, b2, g1, be1,
           g2, be2, Wih, Whh, bih, bhh, Wc1, bc1, Wc2, bc2):
    src = edge_index[0]
    dst = edge_index[1]
    iota_n = jnp.arange(N, dtype=jnp.int32)
    padi = jnp.zeros((E2P - E2,), jnp.int32)
    src2 = jnp.concatenate([src, iota_n, padi]).reshape(E2P // CH, CH)
    dst2 = jnp.concatenate([dst, iota_n, padi]).reshape(E2P // CH, CH)
    ew2 = jnp.concatenate([edge_weight, jnp.ones((N,), jnp.float32),
                           jnp.zeros((E2P - E2,), jnp.float32)]).reshape(E2P // CH, CH)
    x_pad = jnp.pad(x, ((0, NP2 - N), (0, 0)))
    dstd = jnp.pad(dst, (0, EP_DEG - E)).reshape(EP_DEG // CH, CH)
    ewd = jnp.pad(edge_weight, (0, EP_DEG - E)).reshape(EP_DEG // CH, CH)

    zeros1 = jnp.zeros((NP // NSUB,), jnp.float32)
    zeros2 = jnp.zeros((_NR, FH), jnp.float32)

    degp = _deg_kernel(dstd, ewd, zeros1)
    degp_c = degp[:, :NP2].reshape(NCORE, NP2, 1)

    t1a, t1b, dinv_col = pl.pallas_call(
        _prep1_body,
        grid=(8,),
        in_specs=[pl.BlockSpec((NP2 // 8, D_IN), lambda i: (i, 0)),
                  pl.BlockSpec((NCORE, NP2 // 8, 1), lambda i: (0, i, 0)),
                  pl.BlockSpec((D_IN, HID), lambda i: (0, 0))],
        out_specs=[pl.BlockSpec((NP2 // 8, FH), lambda i: (i, 0)),
                   pl.BlockSpec((NP2 // 8, FH), lambda i: (i, 0)),
                   pl.BlockSpec((NP2 // 8, 1), lambda i: (i, 0))],
        out_shape=(jax.ShapeDtypeStruct((NP2, FH), jnp.float32),
                   jax.ShapeDtypeStruct((NP2, FH), jnp.float32),
                   jax.ShapeDtypeStruct((NP2, 1), jnp.float32)),
    )(x_pad, degp_c, W1)

    s1 = _mp_kernel(src2, dst2, ew2, t1a, t1b, zeros2)

    h1, stats1 = _combine_call(s1, dinv_col, b1.reshape(2, FH))

    t2a, t2b = _bnmm_call(h1, stats1, g1, be1, dinv_col, W2)

    s2 = _mp_kernel(src2, dst2, ew2, t2a, t2b, zeros2)

    h2, stats2 = _combine_call(s2, dinv_col, b2.reshape(2, FH))

    logits = _tc_call(
        _poolgru_body,
        jax.ShapeDtypeStruct((1, NCLS), jnp.float32),
        h2, stats2, g2, be2, batch_ids.reshape(N // _POOL_CHUNK, _POOL_CHUNK),
        Wih, Whh, bih, bhh, Wc1, bc1, Wc2, bc2,
        scratch_shapes=[pltpu.VMEM((G, 3 * TDIM), jnp.float32)])
    return logits
